# baseline (device time: 103437 ns/iter reference)
import jax
import jax.numpy as jnp
from jax import lax
from jax.experimental import pallas as pl
from jax.experimental.pallas import tpu as pltpu

N_DEV = 8
SEQ = 2048
CHUNK = 256
D = 1024
NH = 8
DH = 128
SCALE = 0.08838834764831843
QSCALE = SCALE * 1.4426950408889634

F32 = jnp.float32
BF16 = jnp.bfloat16


def kernel(x, Wq, Wo, Wk, Wv):
    x2 = x.reshape(CHUNK, D)

    def body(x_ref, wq_ref, wo_ref, wk_ref, wv_ref, out_ref,
             xg_ref, qf_ref, kf_ref, vf_ref, oblk_ref, part_ref, sbuf_ref,
             ag_send, ag_recv, al_send, al_recv,
             rs_send, rs_recv, ls_send, ls_recv, mid_sem):
        my = lax.axis_index("i")
        right = lax.rem(my + 1, N_DEV)
        left = lax.rem(my + N_DEV - 1, N_DEV)

        barrier = pltpu.get_barrier_semaphore()
        for nbr in (left, right):
            pl.semaphore_signal(barrier, inc=1, device_id=(nbr,),
                                device_id_type=pl.DeviceIdType.MESH)
        pl.semaphore_wait(barrier, 2)

        def qkv_chunk(c):
            row = c * CHUNK
            xb = xg_ref[pl.ds(row, CHUNK), :].astype(F32)
            qf_ref[pl.ds(row, CHUNK), :] = (jnp.dot(
                xb, wq_ref[...], preferred_element_type=F32)
                * QSCALE).astype(BF16)
            kf_ref[pl.ds(row, CHUNK), :] = jnp.dot(
                xb, wk_ref[...], preferred_element_type=F32).astype(BF16)
            vf_ref[pl.ds(row, CHUNK), :] = jnp.dot(
                xb, wv_ref[...], preferred_element_type=F32).astype(BF16)

        HR, HL = 4, 3

        def hop(src_chunk, sems, h, dst):
            d = pltpu.make_async_remote_copy(
                src_ref=xg_ref.at[pl.ds(src_chunk * CHUNK, CHUNK), :],
                dst_ref=xg_ref.at[pl.ds(src_chunk * CHUNK, CHUNK), :],
                send_sem=sems[0].at[h], recv_sem=sems[1].at[h],
                device_id=(dst,), device_id_type=pl.DeviceIdType.MESH,
            )
            d.start()
            return d

        xg_ref[pl.ds(my * CHUNK, CHUNK), :] = x_ref[...].astype(BF16)
        agr = [hop(my, (ag_send, ag_recv), 0, right)]
        agl = [hop(my, (al_send, al_recv), 0, left)]
        qkv_chunk(my)
        for h in range(HR):
            agr[h].wait_recv()
            cr = lax.rem(my - 1 - h + N_DEV, N_DEV)
            if h < HR - 1:
                agr.append(hop(cr, (ag_send, ag_recv), h + 1, right))
            qkv_chunk(cr)
            if h < HL:
                agl[h].wait_recv()
                cl = lax.rem(my + 1 + h, N_DEV)
                if h < HL - 1:
                    agl.append(hop(cl, (al_send, al_recv), h + 1, left))
                qkv_chunk(cl)
        for d in agr + agl:
            d.wait_send()

        for nbr in (left, right):
            pl.semaphore_signal(mid_sem, inc=1, device_id=(nbr,),
                                device_id_type=pl.DeviceIdType.MESH)

        def attn_chunk(c):
            row = c * CHUNK

            def hb(hh, carry):
                hcol = hh * DH
                q = qf_ref[pl.ds(row, CHUNK), pl.ds(hcol, DH)]
                s = lax.dot_general(
                    q, kf_ref[:, pl.ds(hcol, DH)], (((1,), (1,)), ((), ())),
                    preferred_element_type=F32)
                p = jnp.exp2(s.astype(BF16))
                l = jnp.sum(p, axis=1, keepdims=True, dtype=F32)
                ob = jnp.dot(p, vf_ref[:, pl.ds(hcol, DH)],
                             preferred_element_type=F32)
                oblk_ref[:, pl.ds(hcol, DH)] = ob * (1.0 / l)
                return carry

            lax.fori_loop(0, NH, hb, 0, unroll=2)
            part_ref[pl.ds(row, CHUNK), :] = jnp.dot(
                oblk_ref[...], wo_ref[...], preferred_element_type=F32)

        def rs_hop(st, slot, dst, sems):
            r = pltpu.make_async_remote_copy(
                src_ref=sbuf_ref.at[slot],
                dst_ref=xg_ref.at[slot * CHUNK:(slot + 1) * CHUNK, :],
                send_sem=sems[0].at[st], recv_sem=sems[1].at[st],
                device_id=(dst,), device_id_type=pl.DeviceIdType.MESH,
            )
            r.start()
            return r

        def part(c):
            return part_ref[pl.ds(c * CHUNK, CHUNK), :]

        def recv(slot):
            return xg_ref[slot * CHUNK:(slot + 1) * CHUNK, :].astype(F32)

        rsr, rsl = [], []
        for st in range(4):
            cr = lax.rem(my + 4 - st + N_DEV, N_DEV)
            attn_chunk(cr)
            if st < 3:
                cl = lax.rem(my - 3 + st + N_DEV, N_DEV)
                attn_chunk(cl)
            if st == 0:
                pl.semaphore_wait(mid_sem, 2)
                sbuf_ref[0] = part(cr).astype(BF16)
                sbuf_ref[4] = part(cl).astype(BF16)
            else:
                rsr[st - 1].wait_recv()
                sbuf_ref[st] = (part(cr) + recv(st - 1)).astype(BF16)
                if st < 3:
                    rsl[st - 1].wait_recv()
                    sbuf_ref[4 + st] = (part(cl) + recv(4 + st - 1)).astype(BF16)
            rsr.append(rs_hop(st, st, right, (rs_send, rs_recv)))
            if st < 3:
                rsl.append(rs_hop(st, 4 + st, left, (ls_send, ls_recv)))
        attn_chunk(my)
        rsr[3].wait_recv()
        rsl[2].wait_recv()
        out_ref[...] = part(my) + recv(3) + recv(6)
        for r in rsr + rsl:
            r.wait_send()

    out = pl.pallas_call(
        body,
        out_shape=jax.ShapeDtypeStruct((CHUNK, D), jnp.float32),
        in_specs=[pl.BlockSpec(memory_space=pltpu.VMEM)] * 5,
        out_specs=pl.BlockSpec(memory_space=pltpu.VMEM),
        scratch_shapes=[
            pltpu.VMEM((SEQ, D), BF16),
            pltpu.VMEM((SEQ, D), BF16),
            pltpu.VMEM((SEQ, D), BF16),
            pltpu.VMEM((SEQ, D), BF16),
            pltpu.VMEM((CHUNK, D), F32),
            pltpu.VMEM((SEQ, D), F32),
            pltpu.VMEM((N_DEV - 1, CHUNK, D), BF16),
            pltpu.SemaphoreType.DMA((4,)),
            pltpu.SemaphoreType.DMA((4,)),
            pltpu.SemaphoreType.DMA((3,)),
            pltpu.SemaphoreType.DMA((3,)),
            pltpu.SemaphoreType.DMA((4,)),
            pltpu.SemaphoreType.DMA((4,)),
            pltpu.SemaphoreType.DMA((3,)),
            pltpu.SemaphoreType.DMA((3,)),
            pltpu.SemaphoreType.REGULAR,
        ],
        compiler_params=pltpu.CompilerParams(
            collective_id=0, vmem_limit_bytes=100 * 1024 * 1024),
    )(x2, Wq, Wo, Wk, Wv)
    return out.reshape(1, CHUNK, D)


# device time: 96841 ns/iter; 1.0681x vs baseline; 1.0681x over previous
import jax
import jax.numpy as jnp
from jax import lax
from jax.experimental import pallas as pl
from jax.experimental.pallas import tpu as pltpu

N_DEV = 8
SEQ = 2048
CHUNK = 256
D = 1024
NH = 8
DH = 128
SCALE = 0.08838834764831843
QSCALE = SCALE * 1.4426950408889634

F32 = jnp.float32
BF16 = jnp.bfloat16


def kernel(x, Wq, Wo, Wk, Wv):
    x2 = x.reshape(CHUNK, D)

    def body(x_ref, wq_ref, wo_ref, wk_ref, wv_ref, out_ref,
             xg_ref, qf_ref, kf_ref, vf_ref, oblk_ref, part_ref, sbuf_ref,
             ag_send, ag_recv, al_send, al_recv,
             rs_send, rs_recv, ls_send, ls_recv, mid_sem):
        my = lax.axis_index("i")
        right = lax.rem(my + 1, N_DEV)
        left = lax.rem(my + N_DEV - 1, N_DEV)

        barrier = pltpu.get_barrier_semaphore()
        for nbr in (left, right):
            pl.semaphore_signal(barrier, inc=1, device_id=(nbr,),
                                device_id_type=pl.DeviceIdType.MESH)
        pl.semaphore_wait(barrier, 2)

        def qkv_chunk(c):
            row = c * CHUNK
            xb = xg_ref[pl.ds(row, CHUNK), :].astype(F32)
            qf_ref[pl.ds(row, CHUNK), :] = (jnp.dot(
                xb, wq_ref[...], preferred_element_type=F32)
                * QSCALE).astype(BF16)
            kf_ref[pl.ds(row, CHUNK), :] = jnp.dot(
                xb, wk_ref[...], preferred_element_type=F32).astype(BF16)
            vf_ref[pl.ds(row, CHUNK), :] = jnp.dot(
                xb, wv_ref[...], preferred_element_type=F32)

        HR, HL = 4, 3
        HALF = CHUNK // 2

        def hop(src_chunk, half, sems, idx, dst):
            row = src_chunk * CHUNK + half * HALF
            d = pltpu.make_async_remote_copy(
                src_ref=xg_ref.at[pl.ds(row, HALF), :],
                dst_ref=xg_ref.at[pl.ds(row, HALF), :],
                send_sem=sems[0].at[idx], recv_sem=sems[1].at[idx],
                device_id=(dst,), device_id_type=pl.DeviceIdType.MESH,
            )
            d.start()
            return d

        R, L = (ag_send, ag_recv), (al_send, al_recv)
        xg_ref[pl.ds(my * CHUNK, CHUNK), :] = x_ref[...].astype(BF16)
        agr = [hop(my, 0, R, 0, right), hop(my, 1, R, 1, right)]
        agl = [hop(my, 0, L, 0, left), hop(my, 1, L, 1, left)]
        qkv_chunk(my)
        for h in range(HR):
            cr = lax.rem(my - 1 - h + N_DEV, N_DEV)
            agr[2 * h].wait_recv()
            if h < HR - 1:
                agr.append(hop(cr, 0, R, 2 * (h + 1), right))
            agr[2 * h + 1].wait_recv()
            if h < HR - 1:
                agr.append(hop(cr, 1, R, 2 * (h + 1) + 1, right))
            qkv_chunk(cr)
            if h < HL:
                cl = lax.rem(my + 1 + h, N_DEV)
                agl[2 * h].wait_recv()
                if h < HL - 1:
                    agl.append(hop(cl, 0, L, 2 * (h + 1), left))
                agl[2 * h + 1].wait_recv()
                if h < HL - 1:
                    agl.append(hop(cl, 1, L, 2 * (h + 1) + 1, left))
                qkv_chunk(cl)
        for d in agr + agl:
            d.wait_send()

        for nbr in (left, right):
            pl.semaphore_signal(mid_sem, inc=1, device_id=(nbr,),
                                device_id_type=pl.DeviceIdType.MESH)

        def attn_chunk(c):
            row = c * CHUNK

            def hb(hh, carry):
                hcol = hh * DH
                q = qf_ref[pl.ds(row, CHUNK), pl.ds(hcol, DH)]
                s = lax.dot_general(
                    q, kf_ref[:, pl.ds(hcol, DH)], (((1,), (1,)), ((), ())),
                    preferred_element_type=F32)
                p = jnp.exp2(s)
                l = jnp.sum(p, axis=1, keepdims=True)
                ob = jnp.dot(p, vf_ref[:, pl.ds(hcol, DH)],
                             preferred_element_type=F32)
                oblk_ref[:, pl.ds(hcol, DH)] = ob * (1.0 / l)
                return carry

            lax.fori_loop(0, NH, hb, 0, unroll=2)
            part_ref[pl.ds(row, CHUNK), :] = jnp.dot(
                oblk_ref[...], wo_ref[...], preferred_element_type=F32)

        def rs_hop(st, slot, dst, sems):
            r = pltpu.make_async_remote_copy(
                src_ref=sbuf_ref.at[slot],
                dst_ref=xg_ref.at[slot * CHUNK:(slot + 1) * CHUNK, :],
                send_sem=sems[0].at[st], recv_sem=sems[1].at[st],
                device_id=(dst,), device_id_type=pl.DeviceIdType.MESH,
            )
            r.start()
            return r

        def part(c):
            return part_ref[pl.ds(c * CHUNK, CHUNK), :]

        def recv(slot):
            return xg_ref[slot * CHUNK:(slot + 1) * CHUNK, :].astype(F32)

        rsr, rsl = [], []
        for st in range(4):
            cr = lax.rem(my + 4 - st + N_DEV, N_DEV)
            attn_chunk(cr)
            if st < 3:
                cl = lax.rem(my - 3 + st + N_DEV, N_DEV)
                attn_chunk(cl)
            if st == 0:
                pl.semaphore_wait(mid_sem, 2)
                sbuf_ref[0] = part(cr).astype(BF16)
                sbuf_ref[4] = part(cl).astype(BF16)
            else:
                rsr[st - 1].wait_recv()
                sbuf_ref[st] = (part(cr) + recv(st - 1)).astype(BF16)
                if st < 3:
                    rsl[st - 1].wait_recv()
                    sbuf_ref[4 + st] = (part(cl) + recv(4 + st - 1)).astype(BF16)
            rsr.append(rs_hop(st, st, right, (rs_send, rs_recv)))
            if st < 3:
                rsl.append(rs_hop(st, 4 + st, left, (ls_send, ls_recv)))
        attn_chunk(my)
        rsr[3].wait_recv()
        rsl[2].wait_recv()
        out_ref[...] = part(my) + recv(3) + recv(6)
        for r in rsr + rsl:
            r.wait_send()

    out = pl.pallas_call(
        body,
        out_shape=jax.ShapeDtypeStruct((CHUNK, D), jnp.float32),
        in_specs=[pl.BlockSpec(memory_space=pltpu.VMEM)] * 5,
        out_specs=pl.BlockSpec(memory_space=pltpu.VMEM),
        scratch_shapes=[
            pltpu.VMEM((SEQ, D), BF16),
            pltpu.VMEM((SEQ, D), BF16),
            pltpu.VMEM((SEQ, D), BF16),
            pltpu.VMEM((SEQ, D), F32),
            pltpu.VMEM((CHUNK, D), F32),
            pltpu.VMEM((SEQ, D), F32),
            pltpu.VMEM((N_DEV - 1, CHUNK, D), BF16),
            pltpu.SemaphoreType.DMA((8,)),
            pltpu.SemaphoreType.DMA((8,)),
            pltpu.SemaphoreType.DMA((6,)),
            pltpu.SemaphoreType.DMA((6,)),
            pltpu.SemaphoreType.DMA((4,)),
            pltpu.SemaphoreType.DMA((4,)),
            pltpu.SemaphoreType.DMA((3,)),
            pltpu.SemaphoreType.DMA((3,)),
            pltpu.SemaphoreType.REGULAR,
        ],
        compiler_params=pltpu.CompilerParams(
            collective_id=0, vmem_limit_bytes=100 * 1024 * 1024),
    )(x2, Wq, Wo, Wk, Wv)
    return out.reshape(1, CHUNK, D)


# device time: 95296 ns/iter; 1.0854x vs baseline; 1.0162x over previous
import jax
import jax.numpy as jnp
from jax import lax
from jax.experimental import pallas as pl
from jax.experimental.pallas import tpu as pltpu

N_DEV = 8
SEQ = 2048
CHUNK = 256
D = 1024
NH = 8
DH = 128
SCALE = 0.08838834764831843
QSCALE = SCALE * 1.4426950408889634

F32 = jnp.float32
BF16 = jnp.bfloat16


def kernel(x, Wq, Wo, Wk, Wv):
    x2 = x.reshape(CHUNK, D)

    def body(x_ref, wq_ref, wo_ref, wk_ref, wv_ref, out_ref,
             xg_ref, qf_ref, kf_ref, vf_ref, oblk_ref, part_ref, sbuf_ref,
             ag_send, ag_recv, al_send, al_recv,
             rs_send, rs_recv, ls_send, ls_recv, mid_sem):
        my = lax.axis_index("i")
        right = lax.rem(my + 1, N_DEV)
        left = lax.rem(my + N_DEV - 1, N_DEV)

        barrier = pltpu.get_barrier_semaphore()
        for nbr in (left, right):
            pl.semaphore_signal(barrier, inc=1, device_id=(nbr,),
                                device_id_type=pl.DeviceIdType.MESH)
        pl.semaphore_wait(barrier, 2)

        def qkv_chunk(c):
            row = c * CHUNK
            xb = xg_ref[pl.ds(row, CHUNK), :].astype(F32)
            qf_ref[pl.ds(row, CHUNK), :] = (jnp.dot(
                xb, wq_ref[...], preferred_element_type=F32)
                * QSCALE).astype(BF16)
            kf_ref[pl.ds(row, CHUNK), :] = jnp.dot(
                xb, wk_ref[...], preferred_element_type=F32).astype(BF16)
            vf_ref[pl.ds(row, CHUNK), :] = jnp.dot(
                xb, wv_ref[...], preferred_element_type=F32)

        HR, HL = 4, 3
        HALF = CHUNK // 2

        def hop(src_chunk, half, sems, idx, dst):
            row = src_chunk * CHUNK + half * HALF
            d = pltpu.make_async_remote_copy(
                src_ref=xg_ref.at[pl.ds(row, HALF), :],
                dst_ref=xg_ref.at[pl.ds(row, HALF), :],
                send_sem=sems[0].at[idx], recv_sem=sems[1].at[idx],
                device_id=(dst,), device_id_type=pl.DeviceIdType.MESH,
            )
            d.start()
            return d

        R, L = (ag_send, ag_recv), (al_send, al_recv)
        xg_ref[pl.ds(my * CHUNK, CHUNK), :] = x_ref[...].astype(BF16)
        agr = [hop(my, 0, R, 0, right), hop(my, 1, R, 1, right)]
        agl = [hop(my, 0, L, 0, left), hop(my, 1, L, 1, left)]
        qkv_chunk(my)
        for h in range(HR):
            cr = lax.rem(my - 1 - h + N_DEV, N_DEV)
            agr[2 * h].wait_recv()
            if h < HR - 1:
                agr.append(hop(cr, 0, R, 2 * (h + 1), right))
            agr[2 * h + 1].wait_recv()
            if h < HR - 1:
                agr.append(hop(cr, 1, R, 2 * (h + 1) + 1, right))
            qkv_chunk(cr)
            if h < HL:
                cl = lax.rem(my + 1 + h, N_DEV)
                agl[2 * h].wait_recv()
                if h < HL - 1:
                    agl.append(hop(cl, 0, L, 2 * (h + 1), left))
                agl[2 * h + 1].wait_recv()
                if h < HL - 1:
                    agl.append(hop(cl, 1, L, 2 * (h + 1) + 1, left))
                qkv_chunk(cl)
        for d in agr + agl:
            d.wait_send()

        for nbr in (left, right):
            pl.semaphore_signal(mid_sem, inc=1, device_id=(nbr,),
                                device_id_type=pl.DeviceIdType.MESH)

        def attn_chunk(c):
            row = c * CHUNK

            def hb(hh, carry):
                hcol = hh * DH
                q = qf_ref[pl.ds(row, CHUNK), pl.ds(hcol, DH)]
                s = lax.dot_general(
                    q, kf_ref[:, pl.ds(hcol, DH)], (((1,), (1,)), ((), ())),
                    preferred_element_type=F32)
                p = jnp.exp2(s)
                l = jnp.sum(p, axis=1, keepdims=True)
                ob = jnp.dot(p, vf_ref[:, pl.ds(hcol, DH)],
                             preferred_element_type=F32)
                oblk_ref[:, pl.ds(hcol, DH)] = ob * (1.0 / l)
                return carry

            lax.fori_loop(0, NH, hb, 0, unroll=2)
            part_ref[pl.ds(row, CHUNK), :] = jnp.dot(
                oblk_ref[...], wo_ref[...], preferred_element_type=F32)

        def rs_hop(st, slot, dst, sems):
            r = pltpu.make_async_remote_copy(
                src_ref=sbuf_ref.at[slot],
                dst_ref=xg_ref.at[slot * CHUNK:(slot + 1) * CHUNK, :],
                send_sem=sems[0].at[st], recv_sem=sems[1].at[st],
                device_id=(dst,), device_id_type=pl.DeviceIdType.MESH,
            )
            r.start()
            return r

        def part(c):
            return part_ref[pl.ds(c * CHUNK, CHUNK), :]

        def recv(slot):
            return xg_ref[slot * CHUNK:(slot + 1) * CHUNK, :].astype(F32)

        rsr, rsl = [], []
        for st in range(4):
            cr = lax.rem(my + 4 - st + N_DEV, N_DEV)
            attn_chunk(cr)
            if st == 0:
                pl.semaphore_wait(mid_sem, 2)
                sbuf_ref[0] = part(cr).astype(BF16)
            else:
                rsr[st - 1].wait_recv()
                sbuf_ref[st] = (part(cr) + recv(st - 1)).astype(BF16)
            rsr.append(rs_hop(st, st, right, (rs_send, rs_recv)))
            if st < 3:
                cl = lax.rem(my - 3 + st + N_DEV, N_DEV)
                attn_chunk(cl)
                if st == 0:
                    sbuf_ref[4] = part(cl).astype(BF16)
                else:
                    rsl[st - 1].wait_recv()
                    sbuf_ref[4 + st] = (part(cl) + recv(4 + st - 1)).astype(BF16)
                rsl.append(rs_hop(st, 4 + st, left, (ls_send, ls_recv)))
        attn_chunk(my)
        rsr[3].wait_recv()
        rsl[2].wait_recv()
        out_ref[...] = part(my) + recv(3) + recv(6)
        for r in rsr + rsl:
            r.wait_send()

    out = pl.pallas_call(
        body,
        out_shape=jax.ShapeDtypeStruct((CHUNK, D), jnp.float32),
        in_specs=[pl.BlockSpec(memory_space=pltpu.VMEM)] * 5,
        out_specs=pl.BlockSpec(memory_space=pltpu.VMEM),
        scratch_shapes=[
            pltpu.VMEM((SEQ, D), BF16),
            pltpu.VMEM((SEQ, D), BF16),
            pltpu.VMEM((SEQ, D), BF16),
            pltpu.VMEM((SEQ, D), F32),
            pltpu.VMEM((CHUNK, D), F32),
            pltpu.VMEM((SEQ, D), F32),
            pltpu.VMEM((N_DEV - 1, CHUNK, D), BF16),
            pltpu.SemaphoreType.DMA((8,)),
            pltpu.SemaphoreType.DMA((8,)),
            pltpu.SemaphoreType.DMA((6,)),
            pltpu.SemaphoreType.DMA((6,)),
            pltpu.SemaphoreType.DMA((4,)),
            pltpu.SemaphoreType.DMA((4,)),
            pltpu.SemaphoreType.DMA((3,)),
            pltpu.SemaphoreType.DMA((3,)),
            pltpu.SemaphoreType.REGULAR,
        ],
        compiler_params=pltpu.CompilerParams(
            collective_id=0, vmem_limit_bytes=100 * 1024 * 1024),
    )(x2, Wq, Wo, Wk, Wv)
    return out.reshape(1, CHUNK, D)
